# Initial kernel scaffold; baseline (speedup 1.0000x reference)
import jax
import jax.numpy as jnp
from jax.experimental import pallas as pl


def kernel(name_indices, domain_indices, name_table, domain_table, W1, b1, W2, b2):
    raise NotImplementedError("write your pallas kernel here")


# speedup vs baseline: 2.4191x; 2.4191x over previous
import functools

import jax
import jax.numpy as jnp
from jax import lax
from jax.experimental import pallas as pl
from jax.experimental.pallas import tpu as pltpu
from jax.experimental.pallas import tpu_sc as plsc

_NC = 2
_NS = 16


@functools.lru_cache(maxsize=None)
def _make_gather(B, D):
    NW = _NC * _NS
    b_per_w = B // NW
    mesh = plsc.VectorSubcoreMesh(core_axis_name="c", subcore_axis_name="s")

    @functools.partial(
        pl.kernel,
        out_type=(jax.ShapeDtypeStruct((B, D), jnp.float32),
                  jax.ShapeDtypeStruct((B, D), jnp.float32)),
        mesh=mesh,
        scratch_types=[
            pltpu.VMEM((b_per_w,), jnp.int32),
            pltpu.VMEM((b_per_w, D), jnp.float32),
            pltpu.SemaphoreType.DMA,
        ],
    )
    def gather_k(name_tab, dom_tab, name_idx, dom_idx, ne_out, de_out,
                 idx_v, rows_v, sem):
        wid = lax.axis_index("s") * _NC + lax.axis_index("c")
        base = wid * b_per_w
        pltpu.sync_copy(name_idx.at[pl.ds(base, b_per_w)], idx_v)
        pltpu.async_copy(name_tab.at[idx_v], rows_v, sem).wait()
        pltpu.sync_copy(rows_v, ne_out.at[pl.ds(base, b_per_w)])
        pltpu.sync_copy(dom_idx.at[pl.ds(base, b_per_w)], idx_v)
        pltpu.async_copy(dom_tab.at[idx_v], rows_v, sem).wait()
        pltpu.sync_copy(rows_v, de_out.at[pl.ds(base, b_per_w)])

    return gather_k


def _mlp_body(ne_ref, de_ref, w1a_ref, w1b_ref, b1_ref, w2_ref, b2_ref,
              out_ref):
    h = jnp.dot(ne_ref[...], w1a_ref[...], preferred_element_type=jnp.float32)
    h = h + jnp.dot(de_ref[...], w1b_ref[...],
                    preferred_element_type=jnp.float32)
    h = jnp.maximum(h + b1_ref[...], 0.0)
    v = jnp.sum(h * w2_ref[...], axis=1) + b2_ref[0]
    out_ref[...] = 5.0 * jax.nn.sigmoid(v)


@functools.lru_cache(maxsize=None)
def _make_mlp(B, D, H, BLK):
    grid = (B // BLK,)
    return pl.pallas_call(
        _mlp_body,
        grid=grid,
        in_specs=[
            pl.BlockSpec((BLK, D), lambda i: (i, 0)),
            pl.BlockSpec((BLK, D), lambda i: (i, 0)),
            pl.BlockSpec((D, H), lambda i: (0, 0)),
            pl.BlockSpec((D, H), lambda i: (0, 0)),
            pl.BlockSpec((1, H), lambda i: (0, 0)),
            pl.BlockSpec((1, H), lambda i: (0, 0)),
            pl.BlockSpec(memory_space=pltpu.SMEM),
        ],
        out_specs=pl.BlockSpec((BLK,), lambda i: (i,)),
        out_shape=jax.ShapeDtypeStruct((B,), jnp.float32),
    )


def kernel(name_indices, domain_indices, name_table, domain_table,
           W1, b1, W2, b2):
    B = name_indices.shape[0]
    D = name_table.shape[1]
    H = W1.shape[1]
    ne, de = _make_gather(B, D)(
        name_table, domain_table,
        name_indices.astype(jnp.int32), domain_indices.astype(jnp.int32))
    out = _make_mlp(B, D, H, 1024)(
        ne, de, W1[:D], W1[D:], b1.reshape(1, H),
        W2.reshape(1, H), b2)
    return out
